# baseline (speedup 1.0000x reference)
import functools

import jax
import jax.numpy as jnp
from jax import lax
from jax.experimental import pallas as pl
from jax.experimental.pallas import tpu as pltpu
from jax.experimental.pallas import tpu_sc as plsc

_NC = 2
_NS = 16
_NW = _NC * _NS


def _make_sc_gather(B, W):
  assert B % _NW == 0
  bpw = B // _NW
  mesh = plsc.VectorSubcoreMesh(core_axis_name="c", subcore_axis_name="s")

  nch = 4
  ch = bpw // nch

  @functools.partial(
      pl.kernel,
      mesh=mesh,
      out_type=jax.ShapeDtypeStruct((B, W), jnp.float32),
      scratch_types=[
          pltpu.VMEM((bpw,), jnp.int32),
          pltpu.VMEM((ch, W), jnp.float32),
          pltpu.VMEM((ch, W), jnp.float32),
          pltpu.SemaphoreType.DMA,
          pltpu.SemaphoreType.DMA,
      ],
  )
  def gather1(tab_hbm, idx_hbm, out, idx_v, r0, r1, s0, s1):
    wid = lax.axis_index("s") * _NC + lax.axis_index("c")
    base = wid * bpw
    pltpu.sync_copy(idx_hbm.at[pl.ds(base, bpw)], idx_v)
    rows = (r0, r1)
    sems = (s0, s1)
    pend = [None, None]
    for c in range(nch):
      b = c % 2
      pend[b] = pltpu.async_copy(
          tab_hbm.at[idx_v.at[pl.ds(c * ch, ch)]], rows[b], sems[b])
      if c >= 1:
        pb = (c - 1) % 2
        pend[pb].wait()
        pltpu.sync_copy(rows[pb], out.at[pl.ds(base + (c - 1) * ch, ch)])
    lb = (nch - 1) % 2
    pend[lb].wait()
    pltpu.sync_copy(rows[lb], out.at[pl.ds(base + (nch - 1) * ch, ch)])

  return gather1


def _widen_prod_body(mp_ref, ml_ref, pw_ref):
  pw_ref[...] = jnp.concatenate([mp_ref[...], ml_ref[...]], axis=1)


def _widen_user_body(u_ref, uw_ref):
  x = u_ref[...]
  uw_ref[...] = jnp.concatenate([x, jnp.zeros_like(x)], axis=1)


def _widen_prod(mp, ml):
  V, D = mp.shape
  R = 4000
  return pl.pallas_call(
      _widen_prod_body,
      grid=(V // R,),
      in_specs=[pl.BlockSpec((R, D), lambda i: (i, 0))] * 2,
      out_specs=pl.BlockSpec((R, 2 * D), lambda i: (i, 0)),
      out_shape=jax.ShapeDtypeStruct((V, 2 * D), jnp.float32),
  )(mp, ml)


def _widen_user(u):
  V, D = u.shape
  R = 4000
  return pl.pallas_call(
      _widen_user_body,
      grid=(V // R,),
      in_specs=[pl.BlockSpec((R, D), lambda i: (i, 0))],
      out_specs=pl.BlockSpec((R, 2 * D), lambda i: (i, 0)),
      out_shape=jax.ShapeDtypeStruct((V, 2 * D), jnp.float32),
  )(u)


def _ln(x, g, b, eps=1e-5):
  m = jnp.mean(x, axis=-1, keepdims=True)
  xc = x - m
  v = jnp.mean(xc * xc, axis=-1, keepdims=True)
  return xc * lax.rsqrt(v + eps) * g + b


def _dot_t(x, w):
  return lax.dot_general(x, w, (((1,), (1,)), ((), ())),
                         preferred_element_type=jnp.float32)


def _tc_body(u_rows, p_rows,
             mf_gb, mlp_gb, wv, wo, bvo, mf_w,
             fc_w, fc_vecs, l1_w, l1_vecs, l2_w, l2_vecs,
             mlp_w, out_ref):
  mf_g = mf_gb[0:1, :]
  mf_b = mf_gb[1:2, :]
  mlp_g = mlp_gb[0:1, :]
  mlp_b = mlp_gb[1:2, :]

  u_mf = u_rows[:, :64]
  p_mf = p_rows[:, :64]
  p_mlp = p_rows[:, 64:]

  mf_vec = _ln(u_mf, mf_g, mf_b) * _ln(p_mf, mf_g, mf_b)
  mf_term = jnp.sum(mf_vec * mf_w[0:1, :], axis=-1)

  x = _ln(p_mlp, mlp_g, mlp_b)
  a = _dot_t(x, wv[...]) + bvo[0:1, :]
  a = _dot_t(a, wo[...]) + bvo[1:2, :]

  h = _ln(jax.nn.relu(_dot_t(a, fc_w[...]) + fc_vecs[0:1, :]),
          fc_vecs[1:2, :], fc_vecs[2:3, :])
  h = _ln(jax.nn.relu(_dot_t(h, l1_w[...]) + l1_vecs[0:1, :]),
          l1_vecs[1:2, :], l1_vecs[2:3, :])
  h = _ln(jax.nn.relu(_dot_t(h, l2_w[...]) + l2_vecs[0:1, :]),
          l2_vecs[1:2, :], l2_vecs[2:3, :])
  mlp_term = jnp.sum(h * mlp_w[0:1, :], axis=-1)

  logit = mf_term + mlp_term + mlp_w[1, 0]
  out_ref[...] = jax.nn.sigmoid(logit)[:, None]


def _full(shape):
  return pl.BlockSpec(shape, lambda i: (0, 0))


def kernel(params, user_id, product_id):
  p = params
  B = user_id.shape[0]
  D = p["mf_user"].shape[1]

  uid = user_id.astype(jnp.int32)
  pid = product_id.astype(jnp.int32)

  gather = _make_sc_gather(B, 2 * D)
  prod_w = _widen_prod(p["mf_prod"], p["mlp_prod"])
  p_rows = gather(prod_w, pid)
  user_w = _widen_user(p["mf_user"])
  u_rows = gather(user_w, uid)

  a = p["attn"]
  f0 = p["final_W"][0, 0]
  f1 = p["final_W"][0, 1]
  mf_w = (f0 * p["mf_out_W"][0])[None, :]
  bias_total = (f0 * p["mf_out_b"][0] + f1 * p["mlp_out_b"][0]
                + p["final_b"][0])
  mlp_w = jnp.stack([f1 * p["mlp_out_W"][0],
                     jnp.full((D,), bias_total, jnp.float32)])

  mf_gb = jnp.stack([p["mf_g"], p["mf_b"]])
  mlp_gb = jnp.stack([p["mlp_g"], p["mlp_b"]])
  bvo = jnp.stack([a["bv"], a["bo"]])
  H0, H1, H2 = p["fc_W"].shape[0], p["l1_W"].shape[0], p["l2_W"].shape[0]
  fc_w = p["fc_W"][:, :D]
  fc_vecs = jnp.stack([p["fc_b"], p["fc_g"], p["fc_beta"]])
  l1_vecs = jnp.stack([p["l1_b"], p["l1_g"], p["l1_beta"]])
  l2_vecs = jnp.stack([p["l2_b"], p["l2_g"], p["l2_beta"]])

  BLK = 2048
  grid = B // BLK
  row_spec = pl.BlockSpec((BLK, 2 * D), lambda i: (i, 0))

  out = pl.pallas_call(
      _tc_body,
      grid=(grid,),
      in_specs=[
          row_spec, row_spec,
          _full((2, D)), _full((2, D)),
          _full((D, D)), _full((D, D)), _full((2, D)), _full((1, D)),
          _full((H0, D)), _full((3, H0)),
          _full((H1, H0)), _full((3, H1)),
          _full((H2, H1)), _full((3, H2)),
          _full((2, D)),
      ],
      out_specs=pl.BlockSpec((BLK, 1), lambda i: (i, 0)),
      out_shape=jax.ShapeDtypeStruct((B, 1), jnp.float32),
  )(u_rows, p_rows,
    mf_gb, mlp_gb, a["Wv"], a["Wo"], bvo, mf_w,
    fc_w, fc_vecs, p["l1_W"], l1_vecs, p["l2_W"], l2_vecs,
    mlp_w)
  return out


# speedup vs baseline: 1.4666x; 1.4666x over previous
import functools

import jax
import jax.numpy as jnp
from jax import lax
from jax.experimental import pallas as pl
from jax.experimental.pallas import tpu as pltpu
from jax.experimental.pallas import tpu_sc as plsc

_NC = 2
_NS = 16
_NW = _NC * _NS


def _make_sc_gather(B, W):
  assert B % _NW == 0
  bpw = B // _NW
  mesh = plsc.VectorSubcoreMesh(core_axis_name="c", subcore_axis_name="s")

  nch = 4
  ch = bpw // nch

  @functools.partial(
      pl.kernel,
      mesh=mesh,
      out_type=jax.ShapeDtypeStruct((B, W), jnp.float32),
      scratch_types=[
          pltpu.VMEM((bpw,), jnp.int32),
          pltpu.VMEM((ch, W), jnp.float32),
          pltpu.VMEM((ch, W), jnp.float32),
          pltpu.SemaphoreType.DMA,
          pltpu.SemaphoreType.DMA,
      ],
  )
  def gather1(tab_hbm, idx_hbm, out, idx_v, r0, r1, s0, s1):
    wid = lax.axis_index("s") * _NC + lax.axis_index("c")
    base = wid * bpw
    pltpu.sync_copy(idx_hbm.at[pl.ds(base, bpw)], idx_v)
    rows = (r0, r1)
    sems = (s0, s1)
    pend = [None, None]
    for c in range(nch):
      b = c % 2
      pend[b] = pltpu.async_copy(
          tab_hbm.at[idx_v.at[pl.ds(c * ch, ch)]], rows[b], sems[b])
      if c >= 1:
        pb = (c - 1) % 2
        pend[pb].wait()
        pltpu.sync_copy(rows[pb], out.at[pl.ds(base + (c - 1) * ch, ch)])
    lb = (nch - 1) % 2
    pend[lb].wait()
    pltpu.sync_copy(rows[lb], out.at[pl.ds(base + (nch - 1) * ch, ch)])

  return gather1


_WIDEN_C = 2048


def _widen_prod_body(mp_ref, ml_ref, pw_ref):
  cat = jnp.concatenate([mp_ref[...], ml_ref[...]], axis=0)
  i0 = lax.broadcasted_iota(jnp.int32, (128, 128), 0)
  i1 = lax.broadcasted_iota(jnp.int32, (128, 128), 1)
  eye = (i0 == i1).astype(jnp.float32)
  pw_ref[...] = lax.dot_general(cat, eye, (((0,), (0,)), ((), ())),
                                preferred_element_type=jnp.float32)


def _widen_user_body(u_ref, uw_ref):
  i0 = lax.broadcasted_iota(jnp.int32, (64, 128), 0)
  i1 = lax.broadcasted_iota(jnp.int32, (64, 128), 1)
  eye = (i0 == i1).astype(jnp.float32)
  uw_ref[...] = lax.dot_general(u_ref[...], eye, (((0,), (0,)), ((), ())),
                                preferred_element_type=jnp.float32)


def _widen_prod(mpT, mlT):
  D, V = mpT.shape
  C = _WIDEN_C
  return pl.pallas_call(
      _widen_prod_body,
      grid=(pl.cdiv(V, C),),
      in_specs=[pl.BlockSpec((D, C), lambda i: (0, i))] * 2,
      out_specs=pl.BlockSpec((C, 2 * D), lambda i: (i, 0)),
      out_shape=jax.ShapeDtypeStruct((V, 2 * D), jnp.float32),
  )(mpT, mlT)


def _widen_user(uT):
  D, V = uT.shape
  C = _WIDEN_C
  return pl.pallas_call(
      _widen_user_body,
      grid=(pl.cdiv(V, C),),
      in_specs=[pl.BlockSpec((D, C), lambda i: (0, i))],
      out_specs=pl.BlockSpec((C, 2 * D), lambda i: (i, 0)),
      out_shape=jax.ShapeDtypeStruct((V, 2 * D), jnp.float32),
  )(uT)


def _ln(x, g, b, eps=1e-5):
  m = jnp.mean(x, axis=-1, keepdims=True)
  xc = x - m
  v = jnp.mean(xc * xc, axis=-1, keepdims=True)
  return xc * lax.rsqrt(v + eps) * g + b


def _dot_t(x, w):
  return lax.dot_general(x, w, (((1,), (1,)), ((), ())),
                         preferred_element_type=jnp.float32)


def _tc_body(u_rows, p_rows,
             mf_gb, mlp_gb, wv, wo, bvo, mf_w,
             fc_w, fc_vecs, l1_w, l1_vecs, l2_w, l2_vecs,
             mlp_w, out_ref):
  mf_g = mf_gb[0:1, :]
  mf_b = mf_gb[1:2, :]
  mlp_g = mlp_gb[0:1, :]
  mlp_b = mlp_gb[1:2, :]

  u_mf = u_rows[:, :64]
  p_mf = p_rows[:, :64]
  p_mlp = p_rows[:, 64:]

  mf_vec = _ln(u_mf, mf_g, mf_b) * _ln(p_mf, mf_g, mf_b)
  mf_term = jnp.sum(mf_vec * mf_w[0:1, :], axis=-1)

  x = _ln(p_mlp, mlp_g, mlp_b)
  a = _dot_t(x, wv[...]) + bvo[0:1, :]
  a = _dot_t(a, wo[...]) + bvo[1:2, :]

  h = _ln(jax.nn.relu(_dot_t(a, fc_w[...]) + fc_vecs[0:1, :]),
          fc_vecs[1:2, :], fc_vecs[2:3, :])
  h = _ln(jax.nn.relu(_dot_t(h, l1_w[...]) + l1_vecs[0:1, :]),
          l1_vecs[1:2, :], l1_vecs[2:3, :])
  h = _ln(jax.nn.relu(_dot_t(h, l2_w[...]) + l2_vecs[0:1, :]),
          l2_vecs[1:2, :], l2_vecs[2:3, :])
  mlp_term = jnp.sum(h * mlp_w[0:1, :], axis=-1)

  logit = mf_term + mlp_term + mlp_w[1, 0]
  out_ref[...] = jax.nn.sigmoid(logit)[:, None]


def _full(shape):
  return pl.BlockSpec(shape, lambda i: (0, 0))


def kernel(params, user_id, product_id):
  p = params
  B = user_id.shape[0]
  D = p["mf_user"].shape[1]

  uid = user_id.astype(jnp.int32)
  pid = product_id.astype(jnp.int32)

  gather = _make_sc_gather(B, 2 * D)
  prod_w = _widen_prod(p["mf_prod"].T, p["mlp_prod"].T)
  p_rows = gather(prod_w, pid)
  user_w = _widen_user(p["mf_user"].T)
  u_rows = gather(user_w, uid)

  a = p["attn"]
  f0 = p["final_W"][0, 0]
  f1 = p["final_W"][0, 1]
  mf_w = (f0 * p["mf_out_W"][0])[None, :]
  bias_total = (f0 * p["mf_out_b"][0] + f1 * p["mlp_out_b"][0]
                + p["final_b"][0])
  mlp_w = jnp.stack([f1 * p["mlp_out_W"][0],
                     jnp.full((D,), bias_total, jnp.float32)])

  mf_gb = jnp.stack([p["mf_g"], p["mf_b"]])
  mlp_gb = jnp.stack([p["mlp_g"], p["mlp_b"]])
  bvo = jnp.stack([a["bv"], a["bo"]])
  H0, H1, H2 = p["fc_W"].shape[0], p["l1_W"].shape[0], p["l2_W"].shape[0]
  fc_w = p["fc_W"][:, :D]
  fc_vecs = jnp.stack([p["fc_b"], p["fc_g"], p["fc_beta"]])
  l1_vecs = jnp.stack([p["l1_b"], p["l1_g"], p["l1_beta"]])
  l2_vecs = jnp.stack([p["l2_b"], p["l2_g"], p["l2_beta"]])

  BLK = 2048
  grid = B // BLK
  row_spec = pl.BlockSpec((BLK, 2 * D), lambda i: (i, 0))

  out = pl.pallas_call(
      _tc_body,
      grid=(grid,),
      in_specs=[
          row_spec, row_spec,
          _full((2, D)), _full((2, D)),
          _full((D, D)), _full((D, D)), _full((2, D)), _full((1, D)),
          _full((H0, D)), _full((3, H0)),
          _full((H1, H0)), _full((3, H1)),
          _full((H2, H1)), _full((3, H2)),
          _full((2, D)),
      ],
      out_specs=pl.BlockSpec((BLK, 1), lambda i: (i, 0)),
      out_shape=jax.ShapeDtypeStruct((B, 1), jnp.float32),
  )(u_rows, p_rows,
    mf_gb, mlp_gb, a["Wv"], a["Wo"], bvo, mf_w,
    fc_w, fc_vecs, p["l1_W"], l1_vecs, p["l2_W"], l2_vecs,
    mlp_w)
  return out
